# baseline (speedup 1.0000x reference)
import math

import numpy as np

import jax
import jax.numpy as jnp
from jax import lax
from jax.experimental import pallas as pl
from jax.experimental.pallas import tpu as pltpu
from jax.experimental.pallas import tpu_sc as plsc

N = 10000
E = 320000
H = 128
HH = H // 2
HW = HH // 2
EPS = 1e-5

NC = 2
NS = 16
NW = NC * NS
CH = 128
NB = 2
ESL = E // NS
NCH = -(-ESL // CH)
NCHR = -(-NCH // NB) * NB
NCHP = NCHR + NB
NACC = 10240
RPT = NACC // NS
ZR = 16

_TPERM = np.empty((H,), np.int32)
for _h in range(2):
    for _j in range(2):
        for _k in range(16):
            _TPERM[64 * _h + 32 * _j + 2 * _k] = 64 * _h + 32 * _j + _k
            _TPERM[64 * _h + 32 * _j + 2 * _k + 1] = 64 * _h + 32 * _j + 16 + _k
_MASK_HI = -65536


def _sc_body(y_hbm, src_hbm, dst_hbm, out_hbm, cnt_hbm,
             src_v, dst_v, rows_bf, rows_f32, zrow_v, ones_v, zcnt_v,
             acc_sh, cnt_sh, sems):
    cid = lax.axis_index("c")
    sid = lax.axis_index("s")
    wid = cid * NS + sid

    for i in range(ZR):
        for j in range(HH // 16):
            zrow_v[i, pl.ds(j * 16, 16)] = jnp.zeros((16,), jnp.float32)
    for i in range(CH):
        ones_v[i, :] = jnp.ones((16,), jnp.float32)
        zcnt_v[i, :] = jnp.zeros((16,), jnp.float32)

    base = sid * RPT
    for t in range(RPT // ZR):
        pltpu.sync_copy(zrow_v, acc_sh.at[pl.ds(base + t * ZR, ZR)])
    for t in range(RPT // CH):
        pltpu.sync_copy(zcnt_v, cnt_sh.at[pl.ds(base + t * CH, CH)])

    pltpu.sync_copy(src_hbm.at[wid], src_v)
    pltpu.sync_copy(dst_hbm.at[sid], dst_v)
    plsc.subcore_barrier()

    for u in range(NB):
        pltpu.async_copy(y_hbm.at[src_v.at[u]], rows_bf.at[u], sems[u])

    def expand(u):
        def _erow(e, carry):
            for j in range(2):
                w = rows_bf[u, e, pl.ds(16 * j, 16)]
                lo = lax.bitcast_convert_type(w << 16, jnp.float32)
                hi = lax.bitcast_convert_type(w & _MASK_HI, jnp.float32)
                rows_f32[e, pl.ds(32 * j, 16)] = lo
                rows_f32[e, pl.ds(32 * j + 16, 16)] = hi
            return carry

        lax.fori_loop(0, CH, _erow, 0)

    def body(g, carry):
        for u in range(NB):
            ci = g * NB + u
            pltpu.make_async_copy(y_hbm.at[src_v.at[ci]], rows_bf.at[u],
                                  sems[u]).wait()
            expand(u)
            pltpu.async_copy(y_hbm.at[src_v.at[ci + NB]], rows_bf.at[u],
                             sems[u])
            pltpu.sync_copy(rows_f32, acc_sh.at[dst_v.at[ci]], add=True)

            @pl.when(cid == u)
            def _():
                pltpu.sync_copy(ones_v, cnt_sh.at[dst_v.at[ci]], add=True)

        return carry

    lax.fori_loop(0, NCHR // NB, body, 0)
    for u in range(NB):
        pltpu.make_async_copy(y_hbm.at[src_v.at[u]], rows_bf.at[u],
                              sems[u]).wait()
    plsc.subcore_barrier()

    pltpu.sync_copy(acc_sh.at[pl.ds(base, RPT)], out_hbm.at[cid, pl.ds(base, RPT)])
    pltpu.sync_copy(cnt_sh.at[pl.ds(base, RPT)], cnt_hbm.at[cid, pl.ds(base, RPT)])


_sc_mesh = plsc.VectorSubcoreMesh(core_axis_name="c", subcore_axis_name="s")

_sc_agg = pl.kernel(
    _sc_body,
    out_type=(
        jax.ShapeDtypeStruct((NC, NACC, HH), jnp.float32),
        jax.ShapeDtypeStruct((NC, NACC, 16), jnp.float32),
    ),
    mesh=_sc_mesh,
    scratch_types=[
        pltpu.VMEM((NCHP, CH), jnp.int32),
        pltpu.VMEM((NCHR, CH), jnp.int32),
        pltpu.VMEM((NB, CH, HW), jnp.int32),
        pltpu.VMEM((CH, HH), jnp.float32),
        pltpu.VMEM((ZR, HH), jnp.float32),
        pltpu.VMEM((CH, 16), jnp.float32),
        pltpu.VMEM((CH, 16), jnp.float32),
        pltpu.VMEM_SHARED((NACC, HH), jnp.float32),
        pltpu.VMEM_SHARED((NACC, 16), jnp.float32),
        [pltpu.SemaphoreType.DMA] * NB,
    ],
    compiler_params=pltpu.CompilerParams(use_tc_tiling_on_sc=False),
    name="sage_aggregate",
)


def _tc_pre_body(x_ref, wl_ref, wr_ref, b_ref, y_ref, z_ref):
    x = x_ref[...]
    y = jnp.dot(x, wl_ref[...], preferred_element_type=jnp.float32)
    yb = y.astype(jnp.bfloat16)
    y_ref[0, :, :] = yb[:, :HH]
    y_ref[1, :, :] = yb[:, HH:]
    z_ref[...] = jnp.dot(x, wr_ref[...], preferred_element_type=jnp.float32) + b_ref[...]


_tc_pre = pl.pallas_call(
    _tc_pre_body,
    out_shape=(
        jax.ShapeDtypeStruct((NC, N, HH), jnp.bfloat16),
        jax.ShapeDtypeStruct((N, H), jnp.float32),
    ),
)

_SQRT1_2 = 1.0 / math.sqrt(2.0)


def _bn_gelu(out, g, b):
    mean = jnp.mean(out, axis=0, keepdims=True)
    d = out - mean
    var = jnp.mean(d * d, axis=0, keepdims=True)
    nrm = d * lax.rsqrt(var + EPS) * g + b
    return nrm * 0.5 * (1.0 + lax.erf(nrm * _SQRT1_2))


def _agg_combine(p_ref, c_ref, z_ref):
    s = jnp.concatenate([p_ref[0, :N, :], p_ref[1, :N, :]], axis=-1)
    cnt = c_ref[0, :N, 0:1] + c_ref[1, :N, 0:1]
    return s / jnp.maximum(cnt, 1.0) + z_ref[...]


def _tc_mid_body(p_ref, c_ref, z_ref, g_ref, be_ref, wl_ref, wr_ref, b_ref,
                 y_ref, z2_ref):
    h = _bn_gelu(_agg_combine(p_ref, c_ref, z_ref), g_ref[...], be_ref[...])
    y = jnp.dot(h, wl_ref[...], preferred_element_type=jnp.float32)
    yb = y.astype(jnp.bfloat16)
    y_ref[0, :, :] = yb[:, :HH]
    y_ref[1, :, :] = yb[:, HH:]
    z2_ref[...] = jnp.dot(h, wr_ref[...], preferred_element_type=jnp.float32) + b_ref[...]


_tc_mid = pl.pallas_call(
    _tc_mid_body,
    out_shape=(
        jax.ShapeDtypeStruct((NC, N, HH), jnp.bfloat16),
        jax.ShapeDtypeStruct((N, H), jnp.float32),
    ),
)


def _tc_fin_body(p_ref, c_ref, z_ref, g_ref, be_ref, h_ref):
    h_ref[...] = _bn_gelu(_agg_combine(p_ref, c_ref, z_ref), g_ref[...], be_ref[...])


_tc_fin = pl.pallas_call(
    _tc_fin_body,
    out_shape=jax.ShapeDtypeStruct((N, H), jnp.float32),
)


def _pack_words(y_bf):
    return lax.bitcast_convert_type(
        y_bf.reshape(NC * N, HW, 2), jnp.int32)


@jax.jit
def kernel(x, edge_index, W_l0, b_l0, W_r0, gamma0, beta0,
           W_l1, b_l1, W_r1, gamma1, beta1):
    src = edge_index[0]
    dst = edge_index[1]
    tperm = jnp.asarray(_TPERM)
    src16 = jnp.pad(src.reshape(NS, ESL), ((0, 0), (0, NCHP * CH - ESL)))
    dst16 = jnp.pad(dst.reshape(NS, ESL), ((0, 0), (0, NCHR * CH - ESL)),
                    constant_values=N)
    src_p = jnp.stack([src16, src16 + N]).reshape(NW, NCHP, CH)
    dst_p = dst16.reshape(NS, NCHR, CH)

    y0, z0 = _tc_pre(x, W_l0[:, tperm], W_r0, b_l0.reshape(1, H))
    p0, c0 = _sc_agg(_pack_words(y0), src_p, dst_p)
    y1, z1 = _tc_mid(p0, c0, z0, gamma0.reshape(1, H), beta0.reshape(1, H),
                     W_l1[:, tperm], W_r1, b_l1.reshape(1, H))
    p1, _c1 = _sc_agg(_pack_words(y1), src_p, dst_p)
    h = _tc_fin(p1, c0, z1, gamma1.reshape(1, H), beta1.reshape(1, H))
    return h


# speedup vs baseline: 1.3923x; 1.3923x over previous
import math

import jax
import jax.numpy as jnp
from jax import lax
from jax.experimental import pallas as pl
from jax.experimental.pallas import tpu as pltpu
from jax.experimental.pallas import tpu_sc as plsc

N = 10000
E = 320000
H = 128
HH = H // 2
EPS = 1e-5

NC = 2
NS = 16
NW = NC * NS
CH = 128
NB = 1
ESL = E // NS
NCH = -(-ESL // CH)
NCHR = -(-NCH // NB) * NB
NCHP = NCHR + NB
NACC = 10240
RPT = NACC // NS
ZR = 16


def _sc_body(y_hbm, src_hbm, dst_hbm, out_hbm, cnt_hbm,
             src_v, dst_v, rows_v, zrow_v, ones_v, zcnt_v,
             acc_sh, cnt_sh, sems, ssems):
    cid = lax.axis_index("c")
    sid = lax.axis_index("s")
    wid = cid * NS + sid

    for i in range(ZR):
        for j in range(HH // 16):
            zrow_v[i, pl.ds(j * 16, 16)] = jnp.zeros((16,), jnp.float32)
    for i in range(CH):
        ones_v[i, :] = jnp.ones((16,), jnp.float32)
        zcnt_v[i, :] = jnp.zeros((16,), jnp.float32)

    base = sid * RPT
    for t in range(RPT // ZR):
        pltpu.sync_copy(zrow_v, acc_sh.at[pl.ds(base + t * ZR, ZR)])
    for t in range(RPT // CH):
        pltpu.sync_copy(zcnt_v, cnt_sh.at[pl.ds(base + t * CH, CH)])

    pltpu.sync_copy(src_hbm.at[wid], src_v)
    pltpu.sync_copy(dst_hbm.at[sid], dst_v)
    plsc.subcore_barrier()

    def body(ci, carry):
        pltpu.async_copy(y_hbm.at[src_v.at[ci]], rows_v.at[0], sems[0]).wait()
        pltpu.sync_copy(rows_v.at[0], acc_sh.at[dst_v.at[ci]], add=True)

        @pl.when(cid == ci % 2)
        def _():
            pltpu.sync_copy(ones_v, cnt_sh.at[dst_v.at[ci]], add=True)

        return carry

    lax.fori_loop(0, NCHR, body, 0)
    plsc.subcore_barrier()

    pltpu.sync_copy(acc_sh.at[pl.ds(base, RPT)], out_hbm.at[cid, pl.ds(base, RPT)])
    pltpu.sync_copy(cnt_sh.at[pl.ds(base, RPT)], cnt_hbm.at[cid, pl.ds(base, RPT)])


_sc_mesh = plsc.VectorSubcoreMesh(core_axis_name="c", subcore_axis_name="s")

_sc_agg = pl.kernel(
    _sc_body,
    out_type=(
        jax.ShapeDtypeStruct((NC, NACC, HH), jnp.float32),
        jax.ShapeDtypeStruct((NC, NACC, 16), jnp.float32),
    ),
    mesh=_sc_mesh,
    scratch_types=[
        pltpu.VMEM((NCHP, CH), jnp.int32),
        pltpu.VMEM((NCHR, CH), jnp.int32),
        pltpu.VMEM((NB, CH, HH), jnp.float32),
        pltpu.VMEM((ZR, HH), jnp.float32),
        pltpu.VMEM((CH, 16), jnp.float32),
        pltpu.VMEM((CH, 16), jnp.float32),
        pltpu.VMEM_SHARED((NACC, HH), jnp.float32),
        pltpu.VMEM_SHARED((NACC, 16), jnp.float32),
        [pltpu.SemaphoreType.DMA] * NB,
        [pltpu.SemaphoreType.DMA] * NB,
    ],
    compiler_params=pltpu.CompilerParams(use_tc_tiling_on_sc=False),
    name="sage_aggregate",
)


def _tc_pre_body(x_ref, wl_ref, wr_ref, b_ref, y_ref, z_ref):
    x = x_ref[...]
    y = jnp.dot(x, wl_ref[...], preferred_element_type=jnp.float32)
    y_ref[0, :, :] = y[:, :HH]
    y_ref[1, :, :] = y[:, HH:]
    z_ref[...] = jnp.dot(x, wr_ref[...], preferred_element_type=jnp.float32) + b_ref[...]


_tc_pre = pl.pallas_call(
    _tc_pre_body,
    out_shape=(
        jax.ShapeDtypeStruct((NC, N, HH), jnp.float32),
        jax.ShapeDtypeStruct((N, H), jnp.float32),
    ),
)

_SQRT1_2 = 1.0 / math.sqrt(2.0)


def _bn_gelu(out, g, b):
    mean = jnp.mean(out, axis=0, keepdims=True)
    d = out - mean
    var = jnp.mean(d * d, axis=0, keepdims=True)
    nrm = d * lax.rsqrt(var + EPS) * g + b
    return nrm * 0.5 * (1.0 + lax.erf(nrm * _SQRT1_2))


def _agg_combine(p_ref, c_ref, z_ref):
    s = jnp.concatenate([p_ref[0, :N, :], p_ref[1, :N, :]], axis=-1)
    cnt = c_ref[0, :N, 0:1] + c_ref[1, :N, 0:1]
    return s / jnp.maximum(cnt, 1.0) + z_ref[...]


def _tc_mid_body(p_ref, c_ref, z_ref, g_ref, be_ref, wl_ref, wr_ref, b_ref,
                 y_ref, z2_ref):
    h = _bn_gelu(_agg_combine(p_ref, c_ref, z_ref), g_ref[...], be_ref[...])
    y = jnp.dot(h, wl_ref[...], preferred_element_type=jnp.float32)
    y_ref[0, :, :] = y[:, :HH]
    y_ref[1, :, :] = y[:, HH:]
    z2_ref[...] = jnp.dot(h, wr_ref[...], preferred_element_type=jnp.float32) + b_ref[...]


_tc_mid = pl.pallas_call(
    _tc_mid_body,
    out_shape=(
        jax.ShapeDtypeStruct((NC, N, HH), jnp.float32),
        jax.ShapeDtypeStruct((N, H), jnp.float32),
    ),
)


def _tc_fin_body(p_ref, c_ref, z_ref, g_ref, be_ref, h_ref):
    h_ref[...] = _bn_gelu(_agg_combine(p_ref, c_ref, z_ref), g_ref[...], be_ref[...])


_tc_fin = pl.pallas_call(
    _tc_fin_body,
    out_shape=jax.ShapeDtypeStruct((N, H), jnp.float32),
)


@jax.jit
def kernel(x, edge_index, W_l0, b_l0, W_r0, gamma0, beta0,
           W_l1, b_l1, W_r1, gamma1, beta1):
    src = edge_index[0]
    dst = edge_index[1]
    src16 = jnp.pad(src.reshape(NS, ESL), ((0, 0), (0, NCHP * CH - ESL)))
    dst16 = jnp.pad(dst.reshape(NS, ESL), ((0, 0), (0, NCHR * CH - ESL)),
                    constant_values=N)
    src_p = jnp.stack([src16, src16 + N]).reshape(NW, NCHP, CH)
    dst_p = dst16.reshape(NS, NCHR, CH)

    y0, z0 = _tc_pre(x, W_l0, W_r0, b_l0.reshape(1, H))
    p0, c0 = _sc_agg(y0.reshape(NC * N, HH), src_p, dst_p)
    y1, z1 = _tc_mid(p0, c0, z0, gamma0.reshape(1, H), beta0.reshape(1, H),
                     W_l1, W_r1, b_l1.reshape(1, H))
    p1, _c1 = _sc_agg(y1.reshape(NC * N, HH), src_p, dst_p)
    h = _tc_fin(p1, c0, z1, gamma1.reshape(1, H), beta1.reshape(1, H))
    return h
